# baseline (speedup 1.0000x reference)
import functools

import jax
import jax.numpy as jnp
from jax import lax
from jax.experimental import pallas as pl
from jax.experimental.pallas import tpu as pltpu
import jax.experimental.pallas.tpu_sc as plsc

N = 10000
NP = 10240
D = 128
HD = D // 2
E = 320000
NSUB = 16
C = 112
NCH = 184
EPT = NCH * C
PADE = NSUB * EPT
G = 8
NGRP = NCH // G
ROWS_PT = NP // NSUB
R = 1024
NB = NP // R
EPS = 1e-5
MASKHI = -65536


def _sc_mesh():
    return plsc.VectorSubcoreMesh(core_axis_name="c", subcore_axis_name="s")


def _sc_degrees(dst3, w3):
    @functools.partial(
        pl.kernel,
        out_type=jax.ShapeDtypeStruct((2, NSUB, ROWS_PT), jnp.float32),
        mesh=_sc_mesh(),
        scratch_types=[
            pltpu.VMEM((NCH, C), jnp.int32),
            pltpu.VMEM((NCH, C), jnp.float32),
            pltpu.VMEM((ROWS_PT,), jnp.float32),
            pltpu.VMEM_SHARED((NP,), jnp.float32),
        ],
    )
    def run(dst_hbm, w_hbm, deg_out, dst_v, w_v, zero_v, deg_sh):
        c = lax.axis_index("c")
        s = lax.axis_index("s")
        pltpu.sync_copy(dst_hbm.at[c, s], dst_v)
        pltpu.sync_copy(w_hbm.at[c, s], w_v)

        @pl.loop(0, ROWS_PT, step=16)
        def _(i):
            zero_v[pl.ds(i, 16)] = jnp.zeros((16,), jnp.float32)

        pltpu.sync_copy(zero_v, deg_sh.at[pl.ds(s * ROWS_PT, ROWS_PT)])
        plsc.subcore_barrier()

        @pl.loop(0, NCH)
        def _(j):
            pltpu.sync_copy(w_v.at[j], deg_sh.at[dst_v.at[j]], add=True)

        plsc.subcore_barrier()
        pltpu.sync_copy(deg_sh.at[pl.ds(s * ROWS_PT, ROWS_PT)], deg_out.at[c, s])

    return run(dst3, w3)


def _sc_edge_accumulate(src3, dst3, w3, hpp):
    @functools.partial(
        pl.kernel,
        out_type=jax.ShapeDtypeStruct((2, NSUB, ROWS_PT, D), jnp.float32),
        mesh=_sc_mesh(),
        compiler_params=pltpu.CompilerParams(use_tc_tiling_on_sc=False),
        scratch_types=[
            pltpu.VMEM((G, C), jnp.int32),
            pltpu.VMEM((G, C), jnp.int32),
            pltpu.VMEM((G, C), jnp.float32),
            pltpu.VMEM((C, HD), jnp.float32),
            pltpu.VMEM((C, HD), jnp.float32),
            pltpu.VMEM((C, D), jnp.float32),
            pltpu.VMEM((C, D), jnp.float32),
            pltpu.VMEM_SHARED((NP, D), jnp.float32),
            pltpu.SemaphoreType.DMA,
            pltpu.SemaphoreType.DMA,
            pltpu.SemaphoreType.DMA,
            pltpu.SemaphoreType.DMA,
        ],
    )
    def run(src_hbm, dst_hbm, w_hbm, hpp_hbm, a_out,
            src_v, dst_v, w_v, gbuf0, gbuf1, sbuf0, sbuf1, a_sh,
            gsem0, gsem1, ssem0, ssem1):
        c = lax.axis_index("c")
        s = lax.axis_index("s")
        gbuf = (gbuf0, gbuf1)
        sbuf = (sbuf0, sbuf1)
        gsem = (gsem0, gsem1)
        ssem = (ssem0, ssem1)

        @pl.loop(0, C)
        def _(r):
            for k in range(0, D, 16):
                sbuf0[r, pl.ds(k, 16)] = jnp.zeros((16,), jnp.float32)

        for kk in range(ROWS_PT // C):
            pltpu.sync_copy(sbuf0, a_sh.at[pl.ds(s * ROWS_PT + kk * C, C)])
        rem = ROWS_PT - (ROWS_PT // C) * C
        if rem:
            pltpu.sync_copy(
                sbuf0.at[pl.ds(0, rem)],
                a_sh.at[pl.ds(s * ROWS_PT + (ROWS_PT // C) * C, rem)])
        plsc.subcore_barrier()

        def unpack_scale(gb, sb, wrow_ref):
            @pl.loop(0, C, step=16)
            def _(r0):
                wrow = wrow_ref[pl.ds(r0, 16)]
                for t in range(16):
                    wv = jnp.full((16,), wrow[t], jnp.float32)
                    for kk in range(HD // 16):
                        wd = gb[r0 + t, pl.ds(16 * kk, 16)]
                        wi = lax.bitcast_convert_type(wd, jnp.int32)
                        hi = lax.bitcast_convert_type(wi & MASKHI, jnp.float32)
                        lo = lax.bitcast_convert_type(
                            lax.shift_left(wi, 16), jnp.float32)
                        sb[r0 + t, pl.ds(16 * kk, 16)] = hi * wv
                        sb[r0 + t, pl.ds(HD + 16 * kk, 16)] = lo * wv

        @pl.loop(0, NGRP)
        def _(gq):
            pltpu.sync_copy(src_hbm.at[c, s, pl.ds(gq * G, G)], src_v)
            pltpu.sync_copy(dst_hbm.at[c, s, pl.ds(gq * G, G)], dst_v)
            pltpu.sync_copy(w_hbm.at[c, s, pl.ds(gq * G, G)], w_v)

            gd = [None] * G
            sd = [None] * G
            gd[0] = pltpu.async_copy(hpp_hbm.at[src_v.at[0]], gbuf[0], gsem[0])
            for j in range(G):
                b = j % 2
                gd[j].wait()
                if j + 1 < G:
                    gd[j + 1] = pltpu.async_copy(
                        hpp_hbm.at[src_v.at[j + 1]], gbuf[1 - b], gsem[1 - b])
                if j >= 2:
                    sd[j - 2].wait()
                unpack_scale(gbuf[b], sbuf[b], w_v.at[j])
                sd[j] = pltpu.async_copy(
                    sbuf[b], a_sh.at[dst_v.at[j]], ssem[b], add=True)
            sd[G - 2].wait()
            sd[G - 1].wait()

        plsc.subcore_barrier()
        for kk in range(ROWS_PT // C):
            pltpu.sync_copy(a_sh.at[pl.ds(s * ROWS_PT + kk * C, C)],
                            a_out.at[c, s, pl.ds(kk * C, C)])
        if rem:
            pltpu.sync_copy(
                a_sh.at[pl.ds(s * ROWS_PT + (ROWS_PT // C) * C, rem)],
                a_out.at[c, s, pl.ds((ROWS_PT // C) * C, rem)])

    return run(src3, dst3, w3, hpp)


_HI = lax.Precision.HIGHEST


def _pack_rows(h):
    a = lax.bitcast_convert_type(h[:, :HD], jnp.int32) + 0x8000
    b = lax.bitcast_convert_type(h[:, HD:], jnp.int32) + 0x8000
    w = (a & MASKHI) | lax.shift_right_logical(b, 16)
    return lax.bitcast_convert_type(w, jnp.float32)


def _tc_dis_mm0(deg4, x_pad, W0s):
    def body(deg_ref, x_ref, w_ref, hp_ref, hpp_ref, dis_ref):
        deg = deg_ref[...][0, 0, 0] + 1.0
        disb = jnp.where(deg > 0, lax.rsqrt(deg), 0.0)
        dis_ref[...] = disb[None, None, None]
        h = jnp.dot(x_ref[...], w_ref[...][0].T,
                    preferred_element_type=jnp.float32, precision=_HI)
        hp = disb[:, None] * h
        hp_ref[...] = hp[None]
        hpp_ref[...] = _pack_rows(hp)[None]

    return pl.pallas_call(
        body,
        grid=(2, NB),
        in_specs=[
            pl.BlockSpec((1, 1, 1, R), lambda g, i: (g, i, 0, 0)),
            pl.BlockSpec((R, D), lambda g, i: (i, 0)),
            pl.BlockSpec((1, D, D), lambda g, i: (g, 0, 0)),
        ],
        out_specs=[
            pl.BlockSpec((1, R, D), lambda g, i: (g, i, 0)),
            pl.BlockSpec((1, R, HD), lambda g, i: (g, i, 0)),
            pl.BlockSpec((1, 1, 1, R), lambda g, i: (g, i, 0, 0)),
        ],
        out_shape=[
            jax.ShapeDtypeStruct((2, NP, D), jnp.float32),
            jax.ShapeDtypeStruct((2, NP, HD), jnp.float32),
            jax.ShapeDtypeStruct((2, NB, 1, R), jnp.float32),
        ],
    )(deg4, x_pad, W0s)


def _tc_post_stats(A, hp, dis4, bs):
    def body(a_ref, hp_ref, dis_ref, b_ref, y_ref, st_ref):
        i = pl.program_id(1)
        disb = dis_ref[...][0, 0, 0]
        b = b_ref[...][0, 0]
        y = disb[:, None] * (a_ref[...][0] + hp_ref[...][0]) + b[None, :]
        y_ref[...] = y[None]
        rows = i * R + lax.broadcasted_iota(jnp.int32, (R, 1), 0)
        ys = jnp.where(rows < N, y, 0.0)
        st = jnp.stack([jnp.sum(ys, axis=0), jnp.sum(ys * ys, axis=0)])

        @pl.when(i == 0)
        def _():
            st_ref[...] = st[None]

        @pl.when(i > 0)
        def _():
            st_ref[...] += st[None]

    return pl.pallas_call(
        body,
        grid=(2, NB),
        in_specs=[
            pl.BlockSpec((1, R, D), lambda g, i: (g, i, 0)),
            pl.BlockSpec((1, R, D), lambda g, i: (g, i, 0)),
            pl.BlockSpec((1, 1, 1, R), lambda g, i: (g, i, 0, 0)),
            pl.BlockSpec((1, 1, D), lambda g, i: (g, 0, 0)),
        ],
        out_specs=[
            pl.BlockSpec((1, R, D), lambda g, i: (g, i, 0)),
            pl.BlockSpec((1, 2, D), lambda g, i: (g, 0, 0)),
        ],
        out_shape=[
            jax.ShapeDtypeStruct((2, NP, D), jnp.float32),
            jax.ShapeDtypeStruct((2, 2, D), jnp.float32),
        ],
    )(A, hp, dis4, bs)


def _bn_relu(y, st, gb):
    mu = st[0] / N
    var = st[1] / N - mu * mu
    inv = lax.rsqrt(var + EPS)
    return jnp.maximum(gb[0] * (y - mu[None, :]) * inv[None, :] + gb[1], 0.0)


def _tc_bn_mm(y, stats, gbs, W1s, dis4):
    def body(y_ref, st_ref, gb_ref, w_ref, dis_ref, hp_ref, hpp_ref):
        z = _bn_relu(y_ref[...][0], st_ref[...][0], gb_ref[...][0])
        h = jnp.dot(z, w_ref[...][0].T,
                    preferred_element_type=jnp.float32, precision=_HI)
        disb = dis_ref[...][0, 0, 0]
        hp = disb[:, None] * h
        hp_ref[...] = hp[None]
        hpp_ref[...] = _pack_rows(hp)[None]

    return pl.pallas_call(
        body,
        grid=(2, NB),
        in_specs=[
            pl.BlockSpec((1, R, D), lambda g, i: (g, i, 0)),
            pl.BlockSpec((1, 2, D), lambda g, i: (g, 0, 0)),
            pl.BlockSpec((1, 2, D), lambda g, i: (g, 0, 0)),
            pl.BlockSpec((1, D, D), lambda g, i: (g, 0, 0)),
            pl.BlockSpec((1, 1, 1, R), lambda g, i: (g, i, 0, 0)),
        ],
        out_specs=[
            pl.BlockSpec((1, R, D), lambda g, i: (g, i, 0)),
            pl.BlockSpec((1, R, HD), lambda g, i: (g, i, 0)),
        ],
        out_shape=[
            jax.ShapeDtypeStruct((2, NP, D), jnp.float32),
            jax.ShapeDtypeStruct((2, NP, HD), jnp.float32),
        ],
    )(y, stats, gbs, W1s, dis4)


def _tc_gate(y, stats, gbs, Wg2, bg):
    def body(ysc_ref, yfc_ref, st_ref, gb_ref, wg_ref, bg_ref, o_ref):
        st = st_ref[...]
        gb = gb_ref[...]
        z_sc = _bn_relu(ysc_ref[...][0], st[0], gb[0])
        z_fc = _bn_relu(yfc_ref[...][0], st[1], gb[1])
        wg = wg_ref[...]
        logits = (jnp.dot(z_sc, wg[0].T, preferred_element_type=jnp.float32,
                          precision=_HI)
                  + jnp.dot(z_fc, wg[1].T, preferred_element_type=jnp.float32,
                            precision=_HI)
                  + bg_ref[...][0][None, :])
        gate = jax.nn.sigmoid(logits)
        o_ref[...] = gate * z_sc + (1.0 - gate) * z_fc

    return pl.pallas_call(
        body,
        grid=(NB,),
        in_specs=[
            pl.BlockSpec((1, R, D), lambda i: (0, i, 0)),
            pl.BlockSpec((1, R, D), lambda i: (1, i, 0)),
            pl.BlockSpec((2, 2, D), lambda i: (0, 0, 0)),
            pl.BlockSpec((2, 2, D), lambda i: (0, 0, 0)),
            pl.BlockSpec((2, D, D), lambda i: (0, 0, 0)),
            pl.BlockSpec((1, D), lambda i: (0, 0)),
        ],
        out_specs=pl.BlockSpec((R, D), lambda i: (i, 0)),
        out_shape=jax.ShapeDtypeStruct((NP, D), jnp.float32),
    )(y, y, stats, gbs, Wg2, bg)


def _prep_edges(edge_index, edge_weight, g):
    src = edge_index[0]
    dst = edge_index[1]
    pad = PADE - E
    srcp = jnp.concatenate([src, jnp.zeros((pad,), jnp.int32)]) + g * NP
    dstp = jnp.concatenate([dst, jnp.zeros((pad,), jnp.int32)])
    wp = jnp.concatenate([edge_weight, jnp.zeros((pad,), jnp.float32)])
    return (srcp.reshape(NSUB, NCH, C), dstp.reshape(NSUB, NCH, C),
            wp.reshape(NSUB, NCH, C))


def kernel(x, edge_index_sc, edge_weight_sc, edge_index_fc, edge_weight_fc,
           W_sc0, b_sc0, W_sc1, b_sc1, W_fc0, b_fc0, W_fc1, b_fc1,
           g_sc0, be_sc0, g_sc1, be_sc1, g_fc0, be_fc0, g_fc1, be_fc1,
           W_gate, b_gate):
    src_s, dst_s, w_s = _prep_edges(edge_index_sc, edge_weight_sc, 0)
    src_f, dst_f, w_f = _prep_edges(edge_index_fc, edge_weight_fc, 1)
    src3 = jnp.stack([src_s, src_f])
    dst3 = jnp.stack([dst_s, dst_f])
    w3 = jnp.stack([w_s, w_f])

    x_pad = jnp.pad(x, ((0, NP - N), (0, 0)))
    W0s = jnp.stack([W_sc0, W_fc0])
    W1s = jnp.stack([W_sc1, W_fc1])
    b0s = jnp.stack([b_sc0, b_fc0]).reshape(2, 1, D)
    b1s = jnp.stack([b_sc1, b_fc1]).reshape(2, 1, D)
    gb0 = jnp.stack([jnp.stack([g_sc0, be_sc0]), jnp.stack([g_fc0, be_fc0])])
    gb1 = jnp.stack([jnp.stack([g_sc1, be_sc1]), jnp.stack([g_fc1, be_fc1])])
    Wg2 = jnp.stack([W_gate[:, :D], W_gate[:, D:]])

    deg4 = _sc_degrees(dst3, w3).reshape(2, NB, 1, R)

    hp0, hpp0, dis4 = _tc_dis_mm0(deg4, x_pad, W0s)

    A0 = _sc_edge_accumulate(src3, dst3, w3,
                             hpp0.reshape(2 * NP, HD)).reshape(2, NP, D)
    y0, st0 = _tc_post_stats(A0, hp0, dis4, b0s)
    hp1, hpp1 = _tc_bn_mm(y0, st0, gb0, W1s, dis4)

    A1 = _sc_edge_accumulate(src3, dst3, w3,
                             hpp1.reshape(2 * NP, HD)).reshape(2, NP, D)
    y1, st1 = _tc_post_stats(A1, hp1, dis4, b1s)

    out = _tc_gate(y1, st1, gb1, Wg2, b_gate.reshape(1, D))
    return out[:N]


# speedup vs baseline: 1.1070x; 1.1070x over previous
import functools

import jax
import jax.numpy as jnp
from jax import lax
from jax.experimental import pallas as pl
from jax.experimental.pallas import tpu as pltpu
import jax.experimental.pallas.tpu_sc as plsc

N = 10000
NP = 10240
D = 128
HD = D // 2
E = 320000
NSUB = 16
C = 96
NCH = 216
EPT = NCH * C
PADE = NSUB * EPT
G = 8
NGRP = NCH // G
ROWS_PT = NP // NSUB
R = 1024
NB = NP // R
EPS = 1e-5
MASKHI = -65536


def _sc_mesh():
    return plsc.VectorSubcoreMesh(core_axis_name="c", subcore_axis_name="s")


def _sc_degrees(dst3, w3):
    @functools.partial(
        pl.kernel,
        out_type=jax.ShapeDtypeStruct((2, NSUB, ROWS_PT), jnp.float32),
        mesh=_sc_mesh(),
        scratch_types=[
            pltpu.VMEM((NCH, C), jnp.int32),
            pltpu.VMEM((NCH, C), jnp.float32),
            pltpu.VMEM((ROWS_PT,), jnp.float32),
            pltpu.VMEM_SHARED((NP,), jnp.float32),
        ],
    )
    def run(dst_hbm, w_hbm, deg_out, dst_v, w_v, zero_v, deg_sh):
        c = lax.axis_index("c")
        s = lax.axis_index("s")
        pltpu.sync_copy(dst_hbm.at[c, s], dst_v)
        pltpu.sync_copy(w_hbm.at[c, s], w_v)

        @pl.loop(0, ROWS_PT, step=16)
        def _(i):
            zero_v[pl.ds(i, 16)] = jnp.zeros((16,), jnp.float32)

        pltpu.sync_copy(zero_v, deg_sh.at[pl.ds(s * ROWS_PT, ROWS_PT)])
        plsc.subcore_barrier()

        @pl.loop(0, NCH)
        def _(j):
            pltpu.sync_copy(w_v.at[j], deg_sh.at[dst_v.at[j]], add=True)

        plsc.subcore_barrier()
        pltpu.sync_copy(deg_sh.at[pl.ds(s * ROWS_PT, ROWS_PT)], deg_out.at[c, s])

    return run(dst3, w3)


def _sc_edge_accumulate(src3, dst3, w3, hpp):
    @functools.partial(
        pl.kernel,
        out_type=jax.ShapeDtypeStruct((2, NSUB, ROWS_PT, D), jnp.float32),
        mesh=_sc_mesh(),
        compiler_params=pltpu.CompilerParams(use_tc_tiling_on_sc=False),
        scratch_types=[
            pltpu.VMEM((G, C), jnp.int32),
            pltpu.VMEM((G, C), jnp.int32),
            pltpu.VMEM((G, C), jnp.float32),
            pltpu.VMEM((C, HD), jnp.float32),
            pltpu.VMEM((C, HD), jnp.float32),
            pltpu.VMEM((C, HD), jnp.float32),
            pltpu.VMEM((C, D), jnp.float32),
            pltpu.VMEM((C, D), jnp.float32),
            pltpu.VMEM_SHARED((NP, D), jnp.float32),
            pltpu.SemaphoreType.DMA,
            pltpu.SemaphoreType.DMA,
            pltpu.SemaphoreType.DMA,
            pltpu.SemaphoreType.DMA,
            pltpu.SemaphoreType.DMA,
        ],
    )
    def run(src_hbm, dst_hbm, w_hbm, hpp_hbm, a_out,
            src_v, dst_v, w_v, gbuf0, gbuf1, gbuf2, sbuf0, sbuf1, a_sh,
            gsem0, gsem1, gsem2, ssem0, ssem1):
        c = lax.axis_index("c")
        s = lax.axis_index("s")
        gbuf = (gbuf0, gbuf1, gbuf2)
        sbuf = (sbuf0, sbuf1)
        gsem = (gsem0, gsem1, gsem2)
        ssem = (ssem0, ssem1)

        @pl.loop(0, C)
        def _(r):
            for k in range(0, D, 16):
                sbuf0[r, pl.ds(k, 16)] = jnp.zeros((16,), jnp.float32)

        for kk in range(ROWS_PT // C):
            pltpu.sync_copy(sbuf0, a_sh.at[pl.ds(s * ROWS_PT + kk * C, C)])
        rem = ROWS_PT - (ROWS_PT // C) * C
        if rem:
            pltpu.sync_copy(
                sbuf0.at[pl.ds(0, rem)],
                a_sh.at[pl.ds(s * ROWS_PT + (ROWS_PT // C) * C, rem)])
        plsc.subcore_barrier()

        def unpack_scale(gb, sb, wrow_ref):
            @pl.loop(0, C, step=16)
            def _(r0):
                wrow = wrow_ref[pl.ds(r0, 16)]
                for t in range(16):
                    wv = jnp.full((16,), wrow[t], jnp.float32)
                    for kk in range(HD // 16):
                        wd = gb[r0 + t, pl.ds(16 * kk, 16)]
                        wi = lax.bitcast_convert_type(wd, jnp.int32)
                        hi = lax.bitcast_convert_type(wi & MASKHI, jnp.float32)
                        lo = lax.bitcast_convert_type(
                            lax.shift_left(wi, 16), jnp.float32)
                        sb[r0 + t, pl.ds(16 * kk, 16)] = hi * wv
                        sb[r0 + t, pl.ds(HD + 16 * kk, 16)] = lo * wv

        @pl.loop(0, NGRP)
        def _(gq):
            pltpu.sync_copy(src_hbm.at[c, s, pl.ds(gq * G, G)], src_v)
            pltpu.sync_copy(dst_hbm.at[c, s, pl.ds(gq * G, G)], dst_v)
            pltpu.sync_copy(w_hbm.at[c, s, pl.ds(gq * G, G)], w_v)

            gd = [None] * G
            sd = [None] * G
            for p in range(2):
                gd[p] = pltpu.async_copy(
                    hpp_hbm.at[src_v.at[p]], gbuf[p], gsem[p])
            for j in range(G):
                b = j % 2
                gd[j].wait()
                if j + 2 < G:
                    gd[j + 2] = pltpu.async_copy(
                        hpp_hbm.at[src_v.at[j + 2]], gbuf[(j + 2) % 3],
                        gsem[(j + 2) % 3])
                if j >= 2:
                    sd[j - 2].wait()
                unpack_scale(gbuf[j % 3], sbuf[b], w_v.at[j])
                sd[j] = pltpu.async_copy(
                    sbuf[b], a_sh.at[dst_v.at[j]], ssem[b], add=True)
            sd[G - 2].wait()
            sd[G - 1].wait()

        plsc.subcore_barrier()
        for kk in range(ROWS_PT // C):
            pltpu.sync_copy(a_sh.at[pl.ds(s * ROWS_PT + kk * C, C)],
                            a_out.at[c, s, pl.ds(kk * C, C)])
        if rem:
            pltpu.sync_copy(
                a_sh.at[pl.ds(s * ROWS_PT + (ROWS_PT // C) * C, rem)],
                a_out.at[c, s, pl.ds((ROWS_PT // C) * C, rem)])

    return run(src3, dst3, w3, hpp)


_HI = lax.Precision.HIGHEST


def _pack_rows(h):
    a = lax.bitcast_convert_type(h[:, :HD], jnp.int32) + 0x8000
    b = lax.bitcast_convert_type(h[:, HD:], jnp.int32) + 0x8000
    w = (a & MASKHI) | lax.shift_right_logical(b, 16)
    return lax.bitcast_convert_type(w, jnp.float32)


def _tc_dis_mm0(deg4, x_pad, W0s):
    def body(deg_ref, x_ref, w_ref, hp_ref, hpp_ref, dis_ref):
        deg = deg_ref[...][0, 0, 0] + 1.0
        disb = jnp.where(deg > 0, lax.rsqrt(deg), 0.0)
        dis_ref[...] = disb[None, None, None]
        h = jnp.dot(x_ref[...], w_ref[...][0].T,
                    preferred_element_type=jnp.float32, precision=_HI)
        hp = disb[:, None] * h
        hp_ref[...] = hp[None]
        hpp_ref[...] = _pack_rows(hp)[None]

    return pl.pallas_call(
        body,
        grid=(2, NB),
        in_specs=[
            pl.BlockSpec((1, 1, 1, R), lambda g, i: (g, i, 0, 0)),
            pl.BlockSpec((R, D), lambda g, i: (i, 0)),
            pl.BlockSpec((1, D, D), lambda g, i: (g, 0, 0)),
        ],
        out_specs=[
            pl.BlockSpec((1, R, D), lambda g, i: (g, i, 0)),
            pl.BlockSpec((1, R, HD), lambda g, i: (g, i, 0)),
            pl.BlockSpec((1, 1, 1, R), lambda g, i: (g, i, 0, 0)),
        ],
        out_shape=[
            jax.ShapeDtypeStruct((2, NP, D), jnp.float32),
            jax.ShapeDtypeStruct((2, NP, HD), jnp.float32),
            jax.ShapeDtypeStruct((2, NB, 1, R), jnp.float32),
        ],
    )(deg4, x_pad, W0s)


def _tc_post_stats(A, hp, dis4, bs):
    def body(a_ref, hp_ref, dis_ref, b_ref, y_ref, st_ref):
        i = pl.program_id(1)
        disb = dis_ref[...][0, 0, 0]
        b = b_ref[...][0, 0]
        y = disb[:, None] * (a_ref[...][0] + hp_ref[...][0]) + b[None, :]
        y_ref[...] = y[None]
        rows = i * R + lax.broadcasted_iota(jnp.int32, (R, 1), 0)
        ys = jnp.where(rows < N, y, 0.0)
        st = jnp.stack([jnp.sum(ys, axis=0), jnp.sum(ys * ys, axis=0)])

        @pl.when(i == 0)
        def _():
            st_ref[...] = st[None]

        @pl.when(i > 0)
        def _():
            st_ref[...] += st[None]

    return pl.pallas_call(
        body,
        grid=(2, NB),
        in_specs=[
            pl.BlockSpec((1, R, D), lambda g, i: (g, i, 0)),
            pl.BlockSpec((1, R, D), lambda g, i: (g, i, 0)),
            pl.BlockSpec((1, 1, 1, R), lambda g, i: (g, i, 0, 0)),
            pl.BlockSpec((1, 1, D), lambda g, i: (g, 0, 0)),
        ],
        out_specs=[
            pl.BlockSpec((1, R, D), lambda g, i: (g, i, 0)),
            pl.BlockSpec((1, 2, D), lambda g, i: (g, 0, 0)),
        ],
        out_shape=[
            jax.ShapeDtypeStruct((2, NP, D), jnp.float32),
            jax.ShapeDtypeStruct((2, 2, D), jnp.float32),
        ],
    )(A, hp, dis4, bs)


def _bn_relu(y, st, gb):
    mu = st[0] / N
    var = st[1] / N - mu * mu
    inv = lax.rsqrt(var + EPS)
    return jnp.maximum(gb[0] * (y - mu[None, :]) * inv[None, :] + gb[1], 0.0)


def _tc_bn_mm(y, stats, gbs, W1s, dis4):
    def body(y_ref, st_ref, gb_ref, w_ref, dis_ref, hp_ref, hpp_ref):
        z = _bn_relu(y_ref[...][0], st_ref[...][0], gb_ref[...][0])
        h = jnp.dot(z, w_ref[...][0].T,
                    preferred_element_type=jnp.float32, precision=_HI)
        disb = dis_ref[...][0, 0, 0]
        hp = disb[:, None] * h
        hp_ref[...] = hp[None]
        hpp_ref[...] = _pack_rows(hp)[None]

    return pl.pallas_call(
        body,
        grid=(2, NB),
        in_specs=[
            pl.BlockSpec((1, R, D), lambda g, i: (g, i, 0)),
            pl.BlockSpec((1, 2, D), lambda g, i: (g, 0, 0)),
            pl.BlockSpec((1, 2, D), lambda g, i: (g, 0, 0)),
            pl.BlockSpec((1, D, D), lambda g, i: (g, 0, 0)),
            pl.BlockSpec((1, 1, 1, R), lambda g, i: (g, i, 0, 0)),
        ],
        out_specs=[
            pl.BlockSpec((1, R, D), lambda g, i: (g, i, 0)),
            pl.BlockSpec((1, R, HD), lambda g, i: (g, i, 0)),
        ],
        out_shape=[
            jax.ShapeDtypeStruct((2, NP, D), jnp.float32),
            jax.ShapeDtypeStruct((2, NP, HD), jnp.float32),
        ],
    )(y, stats, gbs, W1s, dis4)


def _tc_gate(y, stats, gbs, Wg2, bg):
    def body(ysc_ref, yfc_ref, st_ref, gb_ref, wg_ref, bg_ref, o_ref):
        st = st_ref[...]
        gb = gb_ref[...]
        z_sc = _bn_relu(ysc_ref[...][0], st[0], gb[0])
        z_fc = _bn_relu(yfc_ref[...][0], st[1], gb[1])
        wg = wg_ref[...]
        logits = (jnp.dot(z_sc, wg[0].T, preferred_element_type=jnp.float32,
                          precision=_HI)
                  + jnp.dot(z_fc, wg[1].T, preferred_element_type=jnp.float32,
                            precision=_HI)
                  + bg_ref[...][0][None, :])
        gate = jax.nn.sigmoid(logits)
        o_ref[...] = gate * z_sc + (1.0 - gate) * z_fc

    return pl.pallas_call(
        body,
        grid=(NB,),
        in_specs=[
            pl.BlockSpec((1, R, D), lambda i: (0, i, 0)),
            pl.BlockSpec((1, R, D), lambda i: (1, i, 0)),
            pl.BlockSpec((2, 2, D), lambda i: (0, 0, 0)),
            pl.BlockSpec((2, 2, D), lambda i: (0, 0, 0)),
            pl.BlockSpec((2, D, D), lambda i: (0, 0, 0)),
            pl.BlockSpec((1, D), lambda i: (0, 0)),
        ],
        out_specs=pl.BlockSpec((R, D), lambda i: (i, 0)),
        out_shape=jax.ShapeDtypeStruct((NP, D), jnp.float32),
    )(y, y, stats, gbs, Wg2, bg)


def _prep_edges(edge_index, edge_weight, g):
    src = edge_index[0]
    dst = edge_index[1]
    pad = PADE - E
    srcp = jnp.concatenate([src, jnp.zeros((pad,), jnp.int32)]) + g * NP
    dstp = jnp.concatenate([dst, jnp.zeros((pad,), jnp.int32)])
    wp = jnp.concatenate([edge_weight, jnp.zeros((pad,), jnp.float32)])
    return (srcp.reshape(NSUB, NCH, C), dstp.reshape(NSUB, NCH, C),
            wp.reshape(NSUB, NCH, C))


def kernel(x, edge_index_sc, edge_weight_sc, edge_index_fc, edge_weight_fc,
           W_sc0, b_sc0, W_sc1, b_sc1, W_fc0, b_fc0, W_fc1, b_fc1,
           g_sc0, be_sc0, g_sc1, be_sc1, g_fc0, be_fc0, g_fc1, be_fc1,
           W_gate, b_gate):
    src_s, dst_s, w_s = _prep_edges(edge_index_sc, edge_weight_sc, 0)
    src_f, dst_f, w_f = _prep_edges(edge_index_fc, edge_weight_fc, 1)
    src3 = jnp.stack([src_s, src_f])
    dst3 = jnp.stack([dst_s, dst_f])
    w3 = jnp.stack([w_s, w_f])

    x_pad = jnp.pad(x, ((0, NP - N), (0, 0)))
    W0s = jnp.stack([W_sc0, W_fc0])
    W1s = jnp.stack([W_sc1, W_fc1])
    b0s = jnp.stack([b_sc0, b_fc0]).reshape(2, 1, D)
    b1s = jnp.stack([b_sc1, b_fc1]).reshape(2, 1, D)
    gb0 = jnp.stack([jnp.stack([g_sc0, be_sc0]), jnp.stack([g_fc0, be_fc0])])
    gb1 = jnp.stack([jnp.stack([g_sc1, be_sc1]), jnp.stack([g_fc1, be_fc1])])
    Wg2 = jnp.stack([W_gate[:, :D], W_gate[:, D:]])

    deg4 = _sc_degrees(dst3, w3).reshape(2, NB, 1, R)

    hp0, hpp0, dis4 = _tc_dis_mm0(deg4, x_pad, W0s)

    A0 = _sc_edge_accumulate(src3, dst3, w3,
                             hpp0.reshape(2 * NP, HD)).reshape(2, NP, D)
    y0, st0 = _tc_post_stats(A0, hp0, dis4, b0s)
    hp1, hpp1 = _tc_bn_mm(y0, st0, gb0, W1s, dis4)

    A1 = _sc_edge_accumulate(src3, dst3, w3,
                             hpp1.reshape(2 * NP, HD)).reshape(2, NP, D)
    y1, st1 = _tc_post_stats(A1, hp1, dis4, b1s)

    out = _tc_gate(y1, st1, gb1, Wg2, b_gate.reshape(1, D))
    return out[:N]


# speedup vs baseline: 1.3393x; 1.2099x over previous
import functools

import jax
import jax.numpy as jnp
from jax import lax
from jax.experimental import pallas as pl
from jax.experimental.pallas import tpu as pltpu
import jax.experimental.pallas.tpu_sc as plsc

N = 10000
NP = 10240
D = 128
HD = D // 2
E = 320000
NSUB = 16
C = 96
NCH = 216
EPT = NCH * C
PADE = NSUB * EPT
G = 8
NGRP = NCH // G
ROWS_PT = NP // NSUB
R = 1024
NB = NP // R
EPS = 1e-5
MASKHI = -65536


def _sc_mesh():
    return plsc.VectorSubcoreMesh(core_axis_name="c", subcore_axis_name="s")


def _sc_degrees(dst3, w3):
    @functools.partial(
        pl.kernel,
        out_type=jax.ShapeDtypeStruct((2, NSUB, ROWS_PT), jnp.float32),
        mesh=_sc_mesh(),
        scratch_types=[
            pltpu.VMEM((NCH, C), jnp.int32),
            pltpu.VMEM((NCH, C), jnp.float32),
            pltpu.VMEM((ROWS_PT,), jnp.float32),
            pltpu.VMEM_SHARED((NP,), jnp.float32),
        ],
    )
    def run(dst_hbm, w_hbm, deg_out, dst_v, w_v, zero_v, deg_sh):
        c = lax.axis_index("c")
        s = lax.axis_index("s")
        pltpu.sync_copy(dst_hbm.at[c, s], dst_v)
        pltpu.sync_copy(w_hbm.at[c, s], w_v)

        @pl.loop(0, ROWS_PT, step=16)
        def _(i):
            zero_v[pl.ds(i, 16)] = jnp.zeros((16,), jnp.float32)

        pltpu.sync_copy(zero_v, deg_sh.at[pl.ds(s * ROWS_PT, ROWS_PT)])
        plsc.subcore_barrier()

        @pl.loop(0, NCH)
        def _(j):
            pltpu.sync_copy(w_v.at[j], deg_sh.at[dst_v.at[j]], add=True)

        plsc.subcore_barrier()
        pltpu.sync_copy(deg_sh.at[pl.ds(s * ROWS_PT, ROWS_PT)], deg_out.at[c, s])

    return run(dst3, w3)


def _sc_edge_accumulate(src3, dst3, w3, hpp):
    @functools.partial(
        pl.kernel,
        out_type=jax.ShapeDtypeStruct((2, NSUB, ROWS_PT, D), jnp.float32),
        mesh=_sc_mesh(),
        compiler_params=pltpu.CompilerParams(use_tc_tiling_on_sc=False),
        scratch_types=[
            pltpu.VMEM((G, C), jnp.int32),
            pltpu.VMEM((G, C), jnp.int32),
            pltpu.VMEM((G, C), jnp.float32),
            pltpu.VMEM((C, HD), jnp.float32),
            pltpu.VMEM((C, HD), jnp.float32),
            pltpu.VMEM((C, HD), jnp.float32),
            pltpu.VMEM((C, D), jnp.float32),
            pltpu.VMEM((C, D), jnp.float32),
            pltpu.VMEM_SHARED((NP, D), jnp.float32),
            pltpu.SemaphoreType.DMA,
            pltpu.SemaphoreType.DMA,
            pltpu.SemaphoreType.DMA,
            pltpu.SemaphoreType.DMA,
            pltpu.SemaphoreType.DMA,
        ],
    )
    def run(src_hbm, dst_hbm, w_hbm, hpp_hbm, a_out,
            src_v, dst_v, w_v, gbuf0, gbuf1, gbuf2, sbuf0, sbuf1, a_sh,
            gsem0, gsem1, gsem2, ssem0, ssem1):
        c = lax.axis_index("c")
        s = lax.axis_index("s")
        gbuf = (gbuf0, gbuf1, gbuf2)
        sbuf = (sbuf0, sbuf1)
        gsem = (gsem0, gsem1, gsem2)
        ssem = (ssem0, ssem1)

        @pl.loop(0, C)
        def _(r):
            for k in range(0, D, 16):
                sbuf0[r, pl.ds(k, 16)] = jnp.zeros((16,), jnp.float32)

        for kk in range(ROWS_PT // C):
            pltpu.sync_copy(sbuf0, a_sh.at[pl.ds(s * ROWS_PT + kk * C, C)])
        rem = ROWS_PT - (ROWS_PT // C) * C
        if rem:
            pltpu.sync_copy(
                sbuf0.at[pl.ds(0, rem)],
                a_sh.at[pl.ds(s * ROWS_PT + (ROWS_PT // C) * C, rem)])
        plsc.subcore_barrier()

        def unpack_scale(gb, sb, wrow_ref):
            @plsc.parallel_loop(0, C, step=16, unroll=2)
            def _(r0):
                wrow = wrow_ref[pl.ds(r0, 16)]
                for t in range(16):
                    wv = jnp.full((16,), wrow[t], jnp.float32)
                    for kk in range(HD // 16):
                        wd = gb[r0 + t, pl.ds(16 * kk, 16)]
                        wi = lax.bitcast_convert_type(wd, jnp.int32)
                        hi = lax.bitcast_convert_type(wi & MASKHI, jnp.float32)
                        lo = lax.bitcast_convert_type(
                            lax.shift_left(wi, 16), jnp.float32)
                        sb[r0 + t, pl.ds(16 * kk, 16)] = hi * wv
                        sb[r0 + t, pl.ds(HD + 16 * kk, 16)] = lo * wv

        @pl.loop(0, NGRP)
        def _(gq):
            pltpu.sync_copy(src_hbm.at[c, s, pl.ds(gq * G, G)], src_v)
            pltpu.sync_copy(dst_hbm.at[c, s, pl.ds(gq * G, G)], dst_v)
            pltpu.sync_copy(w_hbm.at[c, s, pl.ds(gq * G, G)], w_v)

            gd = [None] * G
            sd = [None] * G
            for p in range(2):
                gd[p] = pltpu.async_copy(
                    hpp_hbm.at[src_v.at[p]], gbuf[p], gsem[p])
            for j in range(G):
                b = j % 2
                gd[j].wait()
                if j + 2 < G:
                    gd[j + 2] = pltpu.async_copy(
                        hpp_hbm.at[src_v.at[j + 2]], gbuf[(j + 2) % 3],
                        gsem[(j + 2) % 3])
                if j >= 2:
                    sd[j - 2].wait()
                unpack_scale(gbuf[j % 3], sbuf[b], w_v.at[j])
                sd[j] = pltpu.async_copy(
                    sbuf[b], a_sh.at[dst_v.at[j]], ssem[b], add=True)
            sd[G - 2].wait()
            sd[G - 1].wait()

        plsc.subcore_barrier()
        for kk in range(ROWS_PT // C):
            pltpu.sync_copy(a_sh.at[pl.ds(s * ROWS_PT + kk * C, C)],
                            a_out.at[c, s, pl.ds(kk * C, C)])
        if rem:
            pltpu.sync_copy(
                a_sh.at[pl.ds(s * ROWS_PT + (ROWS_PT // C) * C, rem)],
                a_out.at[c, s, pl.ds((ROWS_PT // C) * C, rem)])

    return run(src3, dst3, w3, hpp)


_HI = lax.Precision.HIGHEST


def _pack_rows(h):
    a = lax.bitcast_convert_type(h[:, :HD], jnp.int32) + 0x8000
    b = lax.bitcast_convert_type(h[:, HD:], jnp.int32) + 0x8000
    w = (a & MASKHI) | lax.shift_right_logical(b, 16)
    return lax.bitcast_convert_type(w, jnp.float32)


def _tc_dis_mm0(deg4, x_pad, W0s):
    def body(deg_ref, x_ref, w_ref, hp_ref, hpp_ref, dis_ref):
        deg = deg_ref[...][0, 0, 0] + 1.0
        disb = jnp.where(deg > 0, lax.rsqrt(deg), 0.0)
        dis_ref[...] = disb[None, None, None]
        h = jnp.dot(x_ref[...], w_ref[...][0].T,
                    preferred_element_type=jnp.float32, precision=_HI)
        hp = disb[:, None] * h
        hp_ref[...] = hp[None]
        hpp_ref[...] = _pack_rows(hp)[None]

    return pl.pallas_call(
        body,
        grid=(2, NB),
        in_specs=[
            pl.BlockSpec((1, 1, 1, R), lambda g, i: (g, i, 0, 0)),
            pl.BlockSpec((R, D), lambda g, i: (i, 0)),
            pl.BlockSpec((1, D, D), lambda g, i: (g, 0, 0)),
        ],
        out_specs=[
            pl.BlockSpec((1, R, D), lambda g, i: (g, i, 0)),
            pl.BlockSpec((1, R, HD), lambda g, i: (g, i, 0)),
            pl.BlockSpec((1, 1, 1, R), lambda g, i: (g, i, 0, 0)),
        ],
        out_shape=[
            jax.ShapeDtypeStruct((2, NP, D), jnp.float32),
            jax.ShapeDtypeStruct((2, NP, HD), jnp.float32),
            jax.ShapeDtypeStruct((2, NB, 1, R), jnp.float32),
        ],
    )(deg4, x_pad, W0s)


def _tc_post_stats(A, hp, dis4, bs):
    def body(a_ref, hp_ref, dis_ref, b_ref, y_ref, st_ref):
        i = pl.program_id(1)
        disb = dis_ref[...][0, 0, 0]
        b = b_ref[...][0, 0]
        y = disb[:, None] * (a_ref[...][0] + hp_ref[...][0]) + b[None, :]
        y_ref[...] = y[None]
        rows = i * R + lax.broadcasted_iota(jnp.int32, (R, 1), 0)
        ys = jnp.where(rows < N, y, 0.0)
        st = jnp.stack([jnp.sum(ys, axis=0), jnp.sum(ys * ys, axis=0)])

        @pl.when(i == 0)
        def _():
            st_ref[...] = st[None]

        @pl.when(i > 0)
        def _():
            st_ref[...] += st[None]

    return pl.pallas_call(
        body,
        grid=(2, NB),
        in_specs=[
            pl.BlockSpec((1, R, D), lambda g, i: (g, i, 0)),
            pl.BlockSpec((1, R, D), lambda g, i: (g, i, 0)),
            pl.BlockSpec((1, 1, 1, R), lambda g, i: (g, i, 0, 0)),
            pl.BlockSpec((1, 1, D), lambda g, i: (g, 0, 0)),
        ],
        out_specs=[
            pl.BlockSpec((1, R, D), lambda g, i: (g, i, 0)),
            pl.BlockSpec((1, 2, D), lambda g, i: (g, 0, 0)),
        ],
        out_shape=[
            jax.ShapeDtypeStruct((2, NP, D), jnp.float32),
            jax.ShapeDtypeStruct((2, 2, D), jnp.float32),
        ],
    )(A, hp, dis4, bs)


def _bn_relu(y, st, gb):
    mu = st[0] / N
    var = st[1] / N - mu * mu
    inv = lax.rsqrt(var + EPS)
    return jnp.maximum(gb[0] * (y - mu[None, :]) * inv[None, :] + gb[1], 0.0)


def _tc_bn_mm(y, stats, gbs, W1s, dis4):
    def body(y_ref, st_ref, gb_ref, w_ref, dis_ref, hp_ref, hpp_ref):
        z = _bn_relu(y_ref[...][0], st_ref[...][0], gb_ref[...][0])
        h = jnp.dot(z, w_ref[...][0].T,
                    preferred_element_type=jnp.float32, precision=_HI)
        disb = dis_ref[...][0, 0, 0]
        hp = disb[:, None] * h
        hp_ref[...] = hp[None]
        hpp_ref[...] = _pack_rows(hp)[None]

    return pl.pallas_call(
        body,
        grid=(2, NB),
        in_specs=[
            pl.BlockSpec((1, R, D), lambda g, i: (g, i, 0)),
            pl.BlockSpec((1, 2, D), lambda g, i: (g, 0, 0)),
            pl.BlockSpec((1, 2, D), lambda g, i: (g, 0, 0)),
            pl.BlockSpec((1, D, D), lambda g, i: (g, 0, 0)),
            pl.BlockSpec((1, 1, 1, R), lambda g, i: (g, i, 0, 0)),
        ],
        out_specs=[
            pl.BlockSpec((1, R, D), lambda g, i: (g, i, 0)),
            pl.BlockSpec((1, R, HD), lambda g, i: (g, i, 0)),
        ],
        out_shape=[
            jax.ShapeDtypeStruct((2, NP, D), jnp.float32),
            jax.ShapeDtypeStruct((2, NP, HD), jnp.float32),
        ],
    )(y, stats, gbs, W1s, dis4)


def _tc_gate(y, stats, gbs, Wg2, bg):
    def body(ysc_ref, yfc_ref, st_ref, gb_ref, wg_ref, bg_ref, o_ref):
        st = st_ref[...]
        gb = gb_ref[...]
        z_sc = _bn_relu(ysc_ref[...][0], st[0], gb[0])
        z_fc = _bn_relu(yfc_ref[...][0], st[1], gb[1])
        wg = wg_ref[...]
        logits = (jnp.dot(z_sc, wg[0].T, preferred_element_type=jnp.float32,
                          precision=_HI)
                  + jnp.dot(z_fc, wg[1].T, preferred_element_type=jnp.float32,
                            precision=_HI)
                  + bg_ref[...][0][None, :])
        gate = jax.nn.sigmoid(logits)
        o_ref[...] = gate * z_sc + (1.0 - gate) * z_fc

    return pl.pallas_call(
        body,
        grid=(NB,),
        in_specs=[
            pl.BlockSpec((1, R, D), lambda i: (0, i, 0)),
            pl.BlockSpec((1, R, D), lambda i: (1, i, 0)),
            pl.BlockSpec((2, 2, D), lambda i: (0, 0, 0)),
            pl.BlockSpec((2, 2, D), lambda i: (0, 0, 0)),
            pl.BlockSpec((2, D, D), lambda i: (0, 0, 0)),
            pl.BlockSpec((1, D), lambda i: (0, 0)),
        ],
        out_specs=pl.BlockSpec((R, D), lambda i: (i, 0)),
        out_shape=jax.ShapeDtypeStruct((NP, D), jnp.float32),
    )(y, y, stats, gbs, Wg2, bg)


def _prep_edges(edge_index, edge_weight, g):
    src = edge_index[0]
    dst = edge_index[1]
    pad = PADE - E
    srcp = jnp.concatenate([src, jnp.zeros((pad,), jnp.int32)]) + g * NP
    dstp = jnp.concatenate([dst, jnp.zeros((pad,), jnp.int32)])
    wp = jnp.concatenate([edge_weight, jnp.zeros((pad,), jnp.float32)])
    return (srcp.reshape(NSUB, NCH, C), dstp.reshape(NSUB, NCH, C),
            wp.reshape(NSUB, NCH, C))


def kernel(x, edge_index_sc, edge_weight_sc, edge_index_fc, edge_weight_fc,
           W_sc0, b_sc0, W_sc1, b_sc1, W_fc0, b_fc0, W_fc1, b_fc1,
           g_sc0, be_sc0, g_sc1, be_sc1, g_fc0, be_fc0, g_fc1, be_fc1,
           W_gate, b_gate):
    src_s, dst_s, w_s = _prep_edges(edge_index_sc, edge_weight_sc, 0)
    src_f, dst_f, w_f = _prep_edges(edge_index_fc, edge_weight_fc, 1)
    src3 = jnp.stack([src_s, src_f])
    dst3 = jnp.stack([dst_s, dst_f])
    w3 = jnp.stack([w_s, w_f])

    x_pad = jnp.pad(x, ((0, NP - N), (0, 0)))
    W0s = jnp.stack([W_sc0, W_fc0])
    W1s = jnp.stack([W_sc1, W_fc1])
    b0s = jnp.stack([b_sc0, b_fc0]).reshape(2, 1, D)
    b1s = jnp.stack([b_sc1, b_fc1]).reshape(2, 1, D)
    gb0 = jnp.stack([jnp.stack([g_sc0, be_sc0]), jnp.stack([g_fc0, be_fc0])])
    gb1 = jnp.stack([jnp.stack([g_sc1, be_sc1]), jnp.stack([g_fc1, be_fc1])])
    Wg2 = jnp.stack([W_gate[:, :D], W_gate[:, D:]])

    deg4 = _sc_degrees(dst3, w3).reshape(2, NB, 1, R)

    hp0, hpp0, dis4 = _tc_dis_mm0(deg4, x_pad, W0s)

    A0 = _sc_edge_accumulate(src3, dst3, w3,
                             hpp0.reshape(2 * NP, HD)).reshape(2, NP, D)
    y0, st0 = _tc_post_stats(A0, hp0, dis4, b0s)
    hp1, hpp1 = _tc_bn_mm(y0, st0, gb0, W1s, dis4)

    A1 = _sc_edge_accumulate(src3, dst3, w3,
                             hpp1.reshape(2 * NP, HD)).reshape(2, NP, D)
    y1, st1 = _tc_post_stats(A1, hp1, dis4, b1s)

    out = _tc_gate(y1, st1, gb1, Wg2, b_gate.reshape(1, D))
    return out[:N]


# speedup vs baseline: 1.4019x; 1.0467x over previous
import functools

import jax
import jax.numpy as jnp
from jax import lax
from jax.experimental import pallas as pl
from jax.experimental.pallas import tpu as pltpu
import jax.experimental.pallas.tpu_sc as plsc

N = 10000
NP = 10240
D = 128
HD = D // 2
E = 320000
NSUB = 16
C = 96
NCH = 216
EPT = NCH * C
PADE = NSUB * EPT
G = 12
NGRP = NCH // G
ROWS_PT = NP // NSUB
R = 1024
NB = NP // R
EPS = 1e-5
MASKHI = -65536


def _sc_mesh():
    return plsc.VectorSubcoreMesh(core_axis_name="c", subcore_axis_name="s")


def _sc_degrees(dst3, w3):
    @functools.partial(
        pl.kernel,
        out_type=jax.ShapeDtypeStruct((2, NSUB, ROWS_PT), jnp.float32),
        mesh=_sc_mesh(),
        scratch_types=[
            pltpu.VMEM((NCH, C), jnp.int32),
            pltpu.VMEM((NCH, C), jnp.float32),
            pltpu.VMEM((ROWS_PT,), jnp.float32),
            pltpu.VMEM_SHARED((NP,), jnp.float32),
        ],
    )
    def run(dst_hbm, w_hbm, deg_out, dst_v, w_v, zero_v, deg_sh):
        c = lax.axis_index("c")
        s = lax.axis_index("s")
        pltpu.sync_copy(dst_hbm.at[c, s], dst_v)
        pltpu.sync_copy(w_hbm.at[c, s], w_v)

        @pl.loop(0, ROWS_PT, step=16)
        def _(i):
            zero_v[pl.ds(i, 16)] = jnp.zeros((16,), jnp.float32)

        pltpu.sync_copy(zero_v, deg_sh.at[pl.ds(s * ROWS_PT, ROWS_PT)])
        plsc.subcore_barrier()

        @pl.loop(0, NCH)
        def _(j):
            pltpu.sync_copy(w_v.at[j], deg_sh.at[dst_v.at[j]], add=True)

        plsc.subcore_barrier()
        pltpu.sync_copy(deg_sh.at[pl.ds(s * ROWS_PT, ROWS_PT)], deg_out.at[c, s])

    return run(dst3, w3)


def _sc_edge_accumulate(src3, dst3, w3, hpp):
    @functools.partial(
        pl.kernel,
        out_type=jax.ShapeDtypeStruct((2, NSUB, ROWS_PT, D), jnp.float32),
        mesh=_sc_mesh(),
        compiler_params=pltpu.CompilerParams(use_tc_tiling_on_sc=False),
        scratch_types=[
            pltpu.VMEM((G, C), jnp.int32),
            pltpu.VMEM((G, C), jnp.int32),
            pltpu.VMEM((G, C), jnp.float32),
            pltpu.VMEM((C, HD), jnp.float32),
            pltpu.VMEM((C, HD), jnp.float32),
            pltpu.VMEM((C, HD), jnp.float32),
            pltpu.VMEM((C, D), jnp.float32),
            pltpu.VMEM((C, D), jnp.float32),
            pltpu.VMEM_SHARED((NP, D), jnp.float32),
            pltpu.SemaphoreType.DMA,
            pltpu.SemaphoreType.DMA,
            pltpu.SemaphoreType.DMA,
            pltpu.SemaphoreType.DMA,
            pltpu.SemaphoreType.DMA,
        ],
    )
    def run(src_hbm, dst_hbm, w_hbm, hpp_hbm, a_out,
            src_v, dst_v, w_v, gbuf0, gbuf1, gbuf2, sbuf0, sbuf1, a_sh,
            gsem0, gsem1, gsem2, ssem0, ssem1):
        c = lax.axis_index("c")
        s = lax.axis_index("s")
        gbuf = (gbuf0, gbuf1, gbuf2)
        sbuf = (sbuf0, sbuf1)
        gsem = (gsem0, gsem1, gsem2)
        ssem = (ssem0, ssem1)

        @pl.loop(0, C)
        def _(r):
            for k in range(0, D, 16):
                sbuf0[r, pl.ds(k, 16)] = jnp.zeros((16,), jnp.float32)

        for kk in range(ROWS_PT // C):
            pltpu.sync_copy(sbuf0, a_sh.at[pl.ds(s * ROWS_PT + kk * C, C)])
        rem = ROWS_PT - (ROWS_PT // C) * C
        if rem:
            pltpu.sync_copy(
                sbuf0.at[pl.ds(0, rem)],
                a_sh.at[pl.ds(s * ROWS_PT + (ROWS_PT // C) * C, rem)])
        plsc.subcore_barrier()

        def unpack_scale(gb, sb, wrow_ref):
            @plsc.parallel_loop(0, C, step=16, unroll=2)
            def _(r0):
                wrow = wrow_ref[pl.ds(r0, 16)]
                for t in range(16):
                    wv = jnp.full((16,), wrow[t], jnp.float32)
                    for kk in range(HD // 16):
                        wd = gb[r0 + t, pl.ds(16 * kk, 16)]
                        wi = lax.bitcast_convert_type(wd, jnp.int32)
                        hi = lax.bitcast_convert_type(wi & MASKHI, jnp.float32)
                        lo = lax.bitcast_convert_type(
                            lax.shift_left(wi, 16), jnp.float32)
                        sb[r0 + t, pl.ds(16 * kk, 16)] = hi * wv
                        sb[r0 + t, pl.ds(HD + 16 * kk, 16)] = lo * wv

        @pl.loop(0, NGRP)
        def _(gq):
            pltpu.sync_copy(src_hbm.at[c, s, pl.ds(gq * G, G)], src_v)
            pltpu.sync_copy(dst_hbm.at[c, s, pl.ds(gq * G, G)], dst_v)
            pltpu.sync_copy(w_hbm.at[c, s, pl.ds(gq * G, G)], w_v)

            gd = [None] * G
            sd = [None] * G
            for p in range(2):
                gd[p] = pltpu.async_copy(
                    hpp_hbm.at[src_v.at[p]], gbuf[p], gsem[p])
            for j in range(G):
                b = j % 2
                gd[j].wait()
                if j + 2 < G:
                    gd[j + 2] = pltpu.async_copy(
                        hpp_hbm.at[src_v.at[j + 2]], gbuf[(j + 2) % 3],
                        gsem[(j + 2) % 3])
                if j >= 2:
                    sd[j - 2].wait()
                unpack_scale(gbuf[j % 3], sbuf[b], w_v.at[j])
                sd[j] = pltpu.async_copy(
                    sbuf[b], a_sh.at[dst_v.at[j]], ssem[b], add=True)
            sd[G - 2].wait()
            sd[G - 1].wait()

        plsc.subcore_barrier()
        for kk in range(ROWS_PT // C):
            pltpu.sync_copy(a_sh.at[pl.ds(s * ROWS_PT + kk * C, C)],
                            a_out.at[c, s, pl.ds(kk * C, C)])
        if rem:
            pltpu.sync_copy(
                a_sh.at[pl.ds(s * ROWS_PT + (ROWS_PT // C) * C, rem)],
                a_out.at[c, s, pl.ds((ROWS_PT // C) * C, rem)])

    return run(src3, dst3, w3, hpp)


_HI = lax.Precision.HIGHEST


def _pack_rows(h):
    a = lax.bitcast_convert_type(h[:, :HD], jnp.int32) + 0x8000
    b = lax.bitcast_convert_type(h[:, HD:], jnp.int32) + 0x8000
    w = (a & MASKHI) | lax.shift_right_logical(b, 16)
    return lax.bitcast_convert_type(w, jnp.float32)


def _tc_dis_mm0(deg4, x_pad, W0s):
    def body(deg_ref, x_ref, w_ref, hp_ref, hpp_ref, dis_ref):
        deg = deg_ref[...][0, 0, 0] + 1.0
        disb = jnp.where(deg > 0, lax.rsqrt(deg), 0.0)
        dis_ref[...] = disb[None, None, None]
        h = jnp.dot(x_ref[...], w_ref[...][0].T,
                    preferred_element_type=jnp.float32, precision=_HI)
        hp = disb[:, None] * h
        hp_ref[...] = hp[None]
        hpp_ref[...] = _pack_rows(hp)[None]

    return pl.pallas_call(
        body,
        grid=(2, NB),
        in_specs=[
            pl.BlockSpec((1, 1, 1, R), lambda g, i: (g, i, 0, 0)),
            pl.BlockSpec((R, D), lambda g, i: (i, 0)),
            pl.BlockSpec((1, D, D), lambda g, i: (g, 0, 0)),
        ],
        out_specs=[
            pl.BlockSpec((1, R, D), lambda g, i: (g, i, 0)),
            pl.BlockSpec((1, R, HD), lambda g, i: (g, i, 0)),
            pl.BlockSpec((1, 1, 1, R), lambda g, i: (g, i, 0, 0)),
        ],
        out_shape=[
            jax.ShapeDtypeStruct((2, NP, D), jnp.float32),
            jax.ShapeDtypeStruct((2, NP, HD), jnp.float32),
            jax.ShapeDtypeStruct((2, NB, 1, R), jnp.float32),
        ],
    )(deg4, x_pad, W0s)


def _tc_post_stats(A, hp, dis4, bs):
    def body(a_ref, hp_ref, dis_ref, b_ref, y_ref, st_ref):
        i = pl.program_id(1)
        disb = dis_ref[...][0, 0, 0]
        b = b_ref[...][0, 0]
        y = disb[:, None] * (a_ref[...][0] + hp_ref[...][0]) + b[None, :]
        y_ref[...] = y[None]
        rows = i * R + lax.broadcasted_iota(jnp.int32, (R, 1), 0)
        ys = jnp.where(rows < N, y, 0.0)
        st = jnp.stack([jnp.sum(ys, axis=0), jnp.sum(ys * ys, axis=0)])

        @pl.when(i == 0)
        def _():
            st_ref[...] = st[None]

        @pl.when(i > 0)
        def _():
            st_ref[...] += st[None]

    return pl.pallas_call(
        body,
        grid=(2, NB),
        in_specs=[
            pl.BlockSpec((1, R, D), lambda g, i: (g, i, 0)),
            pl.BlockSpec((1, R, D), lambda g, i: (g, i, 0)),
            pl.BlockSpec((1, 1, 1, R), lambda g, i: (g, i, 0, 0)),
            pl.BlockSpec((1, 1, D), lambda g, i: (g, 0, 0)),
        ],
        out_specs=[
            pl.BlockSpec((1, R, D), lambda g, i: (g, i, 0)),
            pl.BlockSpec((1, 2, D), lambda g, i: (g, 0, 0)),
        ],
        out_shape=[
            jax.ShapeDtypeStruct((2, NP, D), jnp.float32),
            jax.ShapeDtypeStruct((2, 2, D), jnp.float32),
        ],
    )(A, hp, dis4, bs)


def _bn_relu(y, st, gb):
    mu = st[0] / N
    var = st[1] / N - mu * mu
    inv = lax.rsqrt(var + EPS)
    return jnp.maximum(gb[0] * (y - mu[None, :]) * inv[None, :] + gb[1], 0.0)


def _tc_bn_mm(y, stats, gbs, W1s, dis4):
    def body(y_ref, st_ref, gb_ref, w_ref, dis_ref, hp_ref, hpp_ref):
        z = _bn_relu(y_ref[...][0], st_ref[...][0], gb_ref[...][0])
        h = jnp.dot(z, w_ref[...][0].T,
                    preferred_element_type=jnp.float32, precision=_HI)
        disb = dis_ref[...][0, 0, 0]
        hp = disb[:, None] * h
        hp_ref[...] = hp[None]
        hpp_ref[...] = _pack_rows(hp)[None]

    return pl.pallas_call(
        body,
        grid=(2, NB),
        in_specs=[
            pl.BlockSpec((1, R, D), lambda g, i: (g, i, 0)),
            pl.BlockSpec((1, 2, D), lambda g, i: (g, 0, 0)),
            pl.BlockSpec((1, 2, D), lambda g, i: (g, 0, 0)),
            pl.BlockSpec((1, D, D), lambda g, i: (g, 0, 0)),
            pl.BlockSpec((1, 1, 1, R), lambda g, i: (g, i, 0, 0)),
        ],
        out_specs=[
            pl.BlockSpec((1, R, D), lambda g, i: (g, i, 0)),
            pl.BlockSpec((1, R, HD), lambda g, i: (g, i, 0)),
        ],
        out_shape=[
            jax.ShapeDtypeStruct((2, NP, D), jnp.float32),
            jax.ShapeDtypeStruct((2, NP, HD), jnp.float32),
        ],
    )(y, stats, gbs, W1s, dis4)


def _tc_gate(y, stats, gbs, Wg2, bg):
    def body(ysc_ref, yfc_ref, st_ref, gb_ref, wg_ref, bg_ref, o_ref):
        st = st_ref[...]
        gb = gb_ref[...]
        z_sc = _bn_relu(ysc_ref[...][0], st[0], gb[0])
        z_fc = _bn_relu(yfc_ref[...][0], st[1], gb[1])
        wg = wg_ref[...]
        logits = (jnp.dot(z_sc, wg[0].T, preferred_element_type=jnp.float32,
                          precision=_HI)
                  + jnp.dot(z_fc, wg[1].T, preferred_element_type=jnp.float32,
                            precision=_HI)
                  + bg_ref[...][0][None, :])
        gate = jax.nn.sigmoid(logits)
        o_ref[...] = gate * z_sc + (1.0 - gate) * z_fc

    return pl.pallas_call(
        body,
        grid=(NB,),
        in_specs=[
            pl.BlockSpec((1, R, D), lambda i: (0, i, 0)),
            pl.BlockSpec((1, R, D), lambda i: (1, i, 0)),
            pl.BlockSpec((2, 2, D), lambda i: (0, 0, 0)),
            pl.BlockSpec((2, 2, D), lambda i: (0, 0, 0)),
            pl.BlockSpec((2, D, D), lambda i: (0, 0, 0)),
            pl.BlockSpec((1, D), lambda i: (0, 0)),
        ],
        out_specs=pl.BlockSpec((R, D), lambda i: (i, 0)),
        out_shape=jax.ShapeDtypeStruct((NP, D), jnp.float32),
    )(y, y, stats, gbs, Wg2, bg)


def _prep_edges(edge_index, edge_weight, g):
    src = edge_index[0]
    dst = edge_index[1]
    pad = PADE - E
    srcp = jnp.concatenate([src, jnp.zeros((pad,), jnp.int32)]) + g * NP
    dstp = jnp.concatenate([dst, jnp.zeros((pad,), jnp.int32)])
    wp = jnp.concatenate([edge_weight, jnp.zeros((pad,), jnp.float32)])
    return (srcp.reshape(NSUB, NCH, C), dstp.reshape(NSUB, NCH, C),
            wp.reshape(NSUB, NCH, C))


def kernel(x, edge_index_sc, edge_weight_sc, edge_index_fc, edge_weight_fc,
           W_sc0, b_sc0, W_sc1, b_sc1, W_fc0, b_fc0, W_fc1, b_fc1,
           g_sc0, be_sc0, g_sc1, be_sc1, g_fc0, be_fc0, g_fc1, be_fc1,
           W_gate, b_gate):
    src_s, dst_s, w_s = _prep_edges(edge_index_sc, edge_weight_sc, 0)
    src_f, dst_f, w_f = _prep_edges(edge_index_fc, edge_weight_fc, 1)
    src3 = jnp.stack([src_s, src_f])
    dst3 = jnp.stack([dst_s, dst_f])
    w3 = jnp.stack([w_s, w_f])

    x_pad = jnp.pad(x, ((0, NP - N), (0, 0)))
    W0s = jnp.stack([W_sc0, W_fc0])
    W1s = jnp.stack([W_sc1, W_fc1])
    b0s = jnp.stack([b_sc0, b_fc0]).reshape(2, 1, D)
    b1s = jnp.stack([b_sc1, b_fc1]).reshape(2, 1, D)
    gb0 = jnp.stack([jnp.stack([g_sc0, be_sc0]), jnp.stack([g_fc0, be_fc0])])
    gb1 = jnp.stack([jnp.stack([g_sc1, be_sc1]), jnp.stack([g_fc1, be_fc1])])
    Wg2 = jnp.stack([W_gate[:, :D], W_gate[:, D:]])

    deg4 = _sc_degrees(dst3, w3).reshape(2, NB, 1, R)

    hp0, hpp0, dis4 = _tc_dis_mm0(deg4, x_pad, W0s)

    A0 = _sc_edge_accumulate(src3, dst3, w3,
                             hpp0.reshape(2 * NP, HD)).reshape(2, NP, D)
    y0, st0 = _tc_post_stats(A0, hp0, dis4, b0s)
    hp1, hpp1 = _tc_bn_mm(y0, st0, gb0, W1s, dis4)

    A1 = _sc_edge_accumulate(src3, dst3, w3,
                             hpp1.reshape(2 * NP, HD)).reshape(2, NP, D)
    y1, st1 = _tc_post_stats(A1, hp1, dis4, b1s)

    out = _tc_gate(y1, st1, gb1, Wg2, b_gate.reshape(1, D))
    return out[:N]
